# baseline (device time: 101577 ns/iter reference)
import jax
import jax.numpy as jnp
from jax import lax
from jax.experimental import pallas as pl
from jax.experimental.pallas import tpu as pltpu

N_GLOBAL = 4096
EPS = 1e-5
BM = 1024
NSLOT = 4


def kernel(x, gamma, beta):
    m, n_loc = x.shape
    nb = m // BM

    def body(x_ref, g_ref, b_ref, o_ref, send_buf, recv_buf,
             send_sems, recv_sems, credit_sem):
        i = pl.program_id(0)
        p = lax.rem(i, NSLOT)
        my_x = lax.axis_index("x")
        my_y = lax.axis_index("y")
        nbr = (my_x, 1 - my_y)

        xb = x_ref[...]
        s = jnp.sum(xb, axis=1)
        ss = jnp.sum(xb * xb, axis=1)

        @pl.when(i >= NSLOT)
        def _():
            pltpu.make_async_remote_copy(
                src_ref=send_buf.at[p], dst_ref=recv_buf.at[p],
                send_sem=send_sems.at[p], recv_sem=recv_sems.at[p],
                device_id=nbr, device_id_type=pl.DeviceIdType.MESH,
            ).wait_send()
            pl.semaphore_wait(credit_sem, 1)

        send_buf[p, 0, :] = s
        send_buf[p, 1, :] = ss

        @pl.when(i == 0)
        def _():
            barrier = pltpu.get_barrier_semaphore()
            pl.semaphore_signal(
                barrier, inc=1, device_id=nbr,
                device_id_type=pl.DeviceIdType.MESH,
            )
            pl.semaphore_wait(barrier, 1)

        rdma = pltpu.make_async_remote_copy(
            src_ref=send_buf.at[p], dst_ref=recv_buf.at[p],
            send_sem=send_sems.at[p], recv_sem=recv_sems.at[p],
            device_id=nbr, device_id_type=pl.DeviceIdType.MESH,
        )
        rdma.start()
        rdma.wait_recv()

        tot_s = send_buf[p, 0, :] + recv_buf[p, 0, :]
        tot_ss = send_buf[p, 1, :] + recv_buf[p, 1, :]
        mean = tot_s * (1.0 / N_GLOBAL)
        var = tot_ss * (1.0 / N_GLOBAL) - mean * mean
        rstd = lax.rsqrt(var + EPS)
        o_ref[...] = g_ref[...] * ((xb - mean[:, None]) * rstd[:, None]) + b_ref[...]

        @pl.when(i <= nb - 1 - NSLOT)
        def _():
            pl.semaphore_signal(
                credit_sem, inc=1, device_id=nbr,
                device_id_type=pl.DeviceIdType.MESH,
            )

        @pl.when(i == nb - 1)
        def _():
            for q in range(NSLOT):
                pltpu.make_async_remote_copy(
                    src_ref=send_buf.at[q], dst_ref=recv_buf.at[q],
                    send_sem=send_sems.at[q], recv_sem=recv_sems.at[q],
                    device_id=nbr, device_id_type=pl.DeviceIdType.MESH,
                ).wait_send()

    return pl.pallas_call(
        body,
        grid=(nb,),
        in_specs=[
            pl.BlockSpec((BM, n_loc), lambda i: (i, 0)),
            pl.BlockSpec((1, n_loc), lambda i: (0, 0)),
            pl.BlockSpec((1, n_loc), lambda i: (0, 0)),
        ],
        out_specs=pl.BlockSpec((BM, n_loc), lambda i: (i, 0)),
        out_shape=jax.ShapeDtypeStruct((m, n_loc), jnp.float32),
        scratch_shapes=[
            pltpu.VMEM((NSLOT, 2, BM), jnp.float32),
            pltpu.VMEM((NSLOT, 2, BM), jnp.float32),
            pltpu.SemaphoreType.DMA((NSLOT,)),
            pltpu.SemaphoreType.DMA((NSLOT,)),
            pltpu.SemaphoreType.REGULAR,
        ],
        compiler_params=pltpu.CompilerParams(
            collective_id=0, vmem_limit_bytes=48 * 1024 * 1024
        ),
    )(x, gamma.reshape(1, n_loc), beta.reshape(1, n_loc))


# device time: 101250 ns/iter; 1.0032x vs baseline; 1.0032x over previous
import jax
import jax.numpy as jnp
from jax import lax
from jax.experimental import pallas as pl
from jax.experimental.pallas import tpu as pltpu

N_GLOBAL = 4096
EPS = 1e-5
BM = 1024


def kernel(x, gamma, beta):
    m, n_loc = x.shape
    nb = m // BM

    def body(x_ref, g_ref, b_ref, o_ref, send_buf, recv_buf,
             send_sems, recv_sems):
        i = pl.program_id(0)
        my_x = lax.axis_index("x")
        my_y = lax.axis_index("y")
        nbr = (my_x, 1 - my_y)

        xb = x_ref[...]
        send_buf[i, 0, :] = jnp.sum(xb, axis=1)
        send_buf[i, 1, :] = jnp.sum(xb * xb, axis=1)

        @pl.when(i == 0)
        def _():
            barrier = pltpu.get_barrier_semaphore()
            pl.semaphore_signal(
                barrier, inc=1, device_id=nbr,
                device_id_type=pl.DeviceIdType.MESH,
            )
            pl.semaphore_wait(barrier, 1)

        rdma = pltpu.make_async_remote_copy(
            src_ref=send_buf.at[i], dst_ref=recv_buf.at[i],
            send_sem=send_sems.at[i], recv_sem=recv_sems.at[i],
            device_id=nbr, device_id_type=pl.DeviceIdType.MESH,
        )
        rdma.start()
        rdma.wait_recv()

        tot_s = send_buf[i, 0, :] + recv_buf[i, 0, :]
        tot_ss = send_buf[i, 1, :] + recv_buf[i, 1, :]
        mean = tot_s * (1.0 / N_GLOBAL)
        var = tot_ss * (1.0 / N_GLOBAL) - mean * mean
        rstd = lax.rsqrt(var + EPS)
        o_ref[...] = g_ref[...] * ((xb - mean[:, None]) * rstd[:, None]) + b_ref[...]

        @pl.when(i == nb - 1)
        def _():
            for q in range(nb):
                pltpu.make_async_remote_copy(
                    src_ref=send_buf.at[q], dst_ref=recv_buf.at[q],
                    send_sem=send_sems.at[q], recv_sem=recv_sems.at[q],
                    device_id=nbr, device_id_type=pl.DeviceIdType.MESH,
                ).wait_send()

    return pl.pallas_call(
        body,
        grid=(nb,),
        in_specs=[
            pl.BlockSpec((BM, n_loc), lambda i: (i, 0)),
            pl.BlockSpec((1, n_loc), lambda i: (0, 0)),
            pl.BlockSpec((1, n_loc), lambda i: (0, 0)),
        ],
        out_specs=pl.BlockSpec((BM, n_loc), lambda i: (i, 0)),
        out_shape=jax.ShapeDtypeStruct((m, n_loc), jnp.float32),
        scratch_shapes=[
            pltpu.VMEM((nb, 2, BM), jnp.float32),
            pltpu.VMEM((nb, 2, BM), jnp.float32),
            pltpu.SemaphoreType.DMA((nb,)),
            pltpu.SemaphoreType.DMA((nb,)),
        ],
        compiler_params=pltpu.CompilerParams(
            collective_id=0, vmem_limit_bytes=48 * 1024 * 1024
        ),
    )(x, gamma.reshape(1, n_loc), beta.reshape(1, n_loc))


# device time: 72595 ns/iter; 1.3992x vs baseline; 1.3947x over previous
import jax
import jax.numpy as jnp
from jax import lax
from jax.experimental import pallas as pl
from jax.experimental.pallas import tpu as pltpu

N_GLOBAL = 4096
EPS = 1e-5
BM = 1024


def kernel(x, gamma, beta):
    m, n_loc = x.shape
    nb = m // BM

    def stats_body(x_ref, stats_ref, acc_ref, recv_ref, send_sem, recv_sem):
        i = pl.program_id(0)
        xb = x_ref[...]
        acc_ref[0, pl.ds(i * BM, BM)] = jnp.sum(xb, axis=1)
        acc_ref[1, pl.ds(i * BM, BM)] = jnp.sum(xb * xb, axis=1)

        @pl.when(i == nb - 1)
        def _():
            my_x = lax.axis_index("x")
            my_y = lax.axis_index("y")
            nbr = (my_x, 1 - my_y)
            barrier = pltpu.get_barrier_semaphore()
            pl.semaphore_signal(
                barrier, inc=1, device_id=nbr,
                device_id_type=pl.DeviceIdType.MESH,
            )
            pl.semaphore_wait(barrier, 1)
            rdma = pltpu.make_async_remote_copy(
                src_ref=acc_ref,
                dst_ref=recv_ref,
                send_sem=send_sem,
                recv_sem=recv_sem,
                device_id=nbr,
                device_id_type=pl.DeviceIdType.MESH,
            )
            rdma.start()
            rdma.wait()
            tot_s = acc_ref[0, :] + recv_ref[0, :]
            tot_ss = acc_ref[1, :] + recv_ref[1, :]
            mean = tot_s * (1.0 / N_GLOBAL)
            var = tot_ss * (1.0 / N_GLOBAL) - mean * mean
            stats_ref[0, :] = mean
            stats_ref[1, :] = lax.rsqrt(var + EPS)

    stats = pl.pallas_call(
        stats_body,
        grid=(nb,),
        in_specs=[pl.BlockSpec((BM, n_loc), lambda i: (i, 0))],
        out_specs=pl.BlockSpec((2, m), lambda i: (0, 0)),
        out_shape=jax.ShapeDtypeStruct((2, m), jnp.float32),
        scratch_shapes=[
            pltpu.VMEM((2, m), jnp.float32),
            pltpu.VMEM((2, m), jnp.float32),
            pltpu.SemaphoreType.DMA,
            pltpu.SemaphoreType.DMA,
        ],
        compiler_params=pltpu.CompilerParams(
            collective_id=0, vmem_limit_bytes=48 * 1024 * 1024
        ),
    )(x)

    def norm_body(x_ref, g_ref, b_ref, stats_ref, o_ref):
        i = pl.program_id(0)
        mean = stats_ref[0, pl.ds(i * BM, BM)][:, None]
        rstd = stats_ref[1, pl.ds(i * BM, BM)][:, None]
        o_ref[...] = g_ref[...] * ((x_ref[...] - mean) * rstd) + b_ref[...]

    out = pl.pallas_call(
        norm_body,
        grid=(nb,),
        in_specs=[
            pl.BlockSpec((BM, n_loc), lambda i: (i, 0)),
            pl.BlockSpec((1, n_loc), lambda i: (0, 0)),
            pl.BlockSpec((1, n_loc), lambda i: (0, 0)),
            pl.BlockSpec((2, m), lambda i: (0, 0)),
        ],
        out_specs=pl.BlockSpec((BM, n_loc), lambda i: (i, 0)),
        out_shape=jax.ShapeDtypeStruct((m, n_loc), jnp.float32),
        compiler_params=pltpu.CompilerParams(
            vmem_limit_bytes=48 * 1024 * 1024
        ),
    )(x, gamma.reshape(1, n_loc), beta.reshape(1, n_loc), stats)
    return out
